# baseline (speedup 1.0000x reference)
import jax
import jax.numpy as jnp
from jax import lax
from jax.experimental import pallas as pl
from jax.experimental.pallas import tpu as pltpu

M = 1000000
D = 64
B = 16384


def _scatter_body(idx_smem, x_hbm, w_hbm, o_hbm, sem):
    del w_hbm

    def step(p, carry):
        dst = idx_smem[p]
        cp = pltpu.make_async_copy(x_hbm.at[p], o_hbm.at[dst], sem)
        cp.start()
        cp.wait()
        return carry

    lax.fori_loop(0, B, step, 0)


def kernel(idx, x, weight):
    new_weight = pl.pallas_call(
        _scatter_body,
        in_specs=[
            pl.BlockSpec(memory_space=pltpu.SMEM),
            pl.BlockSpec(memory_space=pl.ANY),
            pl.BlockSpec(memory_space=pl.ANY),
        ],
        out_specs=pl.BlockSpec(memory_space=pl.ANY),
        out_shape=jax.ShapeDtypeStruct((M, D), weight.dtype),
        scratch_shapes=[pltpu.SemaphoreType.DMA],
        input_output_aliases={2: 0},
    )(idx, x, weight)
    return (x, new_weight)


# speedup vs baseline: 17.3958x; 17.3958x over previous
import functools

import jax
import jax.numpy as jnp
from jax import lax
from jax.experimental import pallas as pl
from jax.experimental.pallas import tpu as pltpu
from jax.experimental.pallas import tpu_sc as plsc

M = 1000000
D = 64
B = 16384
NC = 2
NS = 16
NW = NC * NS
RPT = M // NW
RPT_PAD = 31280

_mesh = plsc.VectorSubcoreMesh(core_axis_name="c", subcore_axis_name="s")


@functools.partial(
    pl.kernel,
    mesh=_mesh,
    compiler_params=pltpu.CompilerParams(needs_layout_passes=False),
    scratch_types=[
        pltpu.VMEM((B,), jnp.int32),
        pltpu.VMEM((B + 16,), jnp.int32),
        pltpu.VMEM((B + 16,), jnp.int32),
        pltpu.VMEM((RPT_PAD,), jnp.int32),
        pltpu.SemaphoreType.DMA,
    ],
)
def _sc_scatter(idx_hbm, x_hbm, w_ref, idx_v, dst_l, pos_l, seen, ssem):
    wid = lax.axis_index("s") * NC + lax.axis_index("c")
    lo = wid * RPT
    hi = lo + RPT

    pltpu.sync_copy(idx_hbm, idx_v)

    zeros16 = jnp.zeros((16,), jnp.int32)

    def zero_body(i, carry):
        seen[pl.ds(i * 16, 16)] = zeros16
        return carry

    lax.fori_loop(0, RPT_PAD // 16, zero_body, 0)

    lanes = lax.iota(jnp.int32, 16)

    def scan_body(i, cnt):
        base = i * 16
        v = idx_v[pl.ds(base, 16)]
        m = (v >= lo) & (v < hi)
        mi = m.astype(jnp.int32)
        incl = plsc.cumsum(mi)
        off = cnt + incl - mi
        plsc.store_scatter(dst_l, [off], v, mask=m)
        plsc.store_scatter(pos_l, [off], base + lanes, mask=m)
        return cnt + incl[15]

    cnt = lax.fori_loop(0, B // 16, scan_body, jnp.int32(0))

    lane0 = lanes == 0
    zvec = jnp.zeros((16,), jnp.int32)
    W = 16

    def drain_one():
        pltpu.make_async_copy(x_hbm.at[0], w_ref.at[0], ssem).wait()

    def fire(k):
        p = cnt - 1 - k
        r = dst_l[pl.ds(p, 16)][0]
        rr = r - lo
        s = seen[pl.ds(rr, 16)][0]
        cand = pos_l[pl.ds(p, 16)][0]
        winner = jnp.where(s == 0, cand, s - 1)
        plsc.store_scatter(seen, [zvec + rr], zvec + winner + 1, mask=lane0)
        pltpu.async_copy(x_hbm.at[winner], w_ref.at[r], ssem)

    def fire_body(k, carry):
        fire(k)
        return carry

    def fire_drain_body(k, carry):
        fire(k)
        drain_one()
        return carry

    head = jnp.minimum(cnt, W)
    lax.fori_loop(0, head, fire_body, 0)
    lax.fori_loop(head, cnt, fire_drain_body, 0)

    def drain_body(i, carry):
        drain_one()
        return carry

    lax.fori_loop(0, head, drain_body, 0)


def kernel(idx, x, weight):
    w2 = jax.new_ref(weight)
    _sc_scatter(idx, x, w2)
    return (x, w2[...])
